# Initial kernel scaffold; baseline (speedup 1.0000x reference)
import jax
import jax.numpy as jnp
from jax.experimental import pallas as pl


def kernel(node_feats, emb_weight, lin_W, lin_b, batch):
    raise NotImplementedError("write your pallas kernel here")


# speedup vs baseline: 2.7974x; 2.7974x over previous
import functools

import jax
import jax.numpy as jnp
from jax import lax
from jax.experimental import pallas as pl
from jax.experimental.pallas import tpu as pltpu
from jax.experimental.pallas import tpu_sc as plsc

N = 100000
D_FEAT = 128
EMB = 128
NF_EMB = 64
B = 16384

NUM_CORES = 2
NUM_SUBCORES = 16
NUM_WORKERS = NUM_CORES * NUM_SUBCORES
B_PER_W = B // NUM_WORKERS


def _sc_gather2(emb_weight, node_feats, batch):
    mesh = plsc.VectorSubcoreMesh(core_axis_name="c", subcore_axis_name="s")

    @functools.partial(
        pl.kernel,
        mesh=mesh,
        out_type=(
            jax.ShapeDtypeStruct((B, EMB), jnp.float32),
            jax.ShapeDtypeStruct((B, D_FEAT), jnp.float32),
        ),
        scratch_types=[
            pltpu.VMEM((B_PER_W,), jnp.int32),
            pltpu.VMEM((B_PER_W, EMB), jnp.float32),
            pltpu.SemaphoreType.DMA,
        ],
    )
    def k(emb_hbm, nf_hbm, idx_hbm, emb_out, nf_out, idx_v, rows_v, sem):
        wid = lax.axis_index("s") * NUM_CORES + lax.axis_index("c")
        base = wid * B_PER_W
        pltpu.sync_copy(idx_hbm.at[pl.ds(base, B_PER_W)], idx_v)
        pltpu.async_copy(emb_hbm.at[idx_v], rows_v, sem).wait()
        pltpu.sync_copy(rows_v, emb_out.at[pl.ds(base, B_PER_W)])
        pltpu.async_copy(nf_hbm.at[idx_v], rows_v, sem).wait()
        pltpu.sync_copy(rows_v, nf_out.at[pl.ds(base, B_PER_W)])

    return k(emb_weight, node_feats, batch)


def _tc_fuse_kernel(emb_ref, nf_ref, w_ref, b_ref, out_ref):
    z = jnp.dot(nf_ref[...], w_ref[...], preferred_element_type=jnp.float32)
    out_ref[:, :EMB] = emb_ref[...]
    out_ref[:, EMB:] = z + b_ref[...]


def _tc_fuse(emb_rows, nf_rows, lin_W, lin_b):
    bm = 2048
    grid = (B // bm,)
    return pl.pallas_call(
        _tc_fuse_kernel,
        grid=grid,
        in_specs=[
            pl.BlockSpec((bm, EMB), lambda i: (i, 0)),
            pl.BlockSpec((bm, D_FEAT), lambda i: (i, 0)),
            pl.BlockSpec((D_FEAT, NF_EMB), lambda i: (0, 0)),
            pl.BlockSpec((1, NF_EMB), lambda i: (0, 0)),
        ],
        out_specs=pl.BlockSpec((bm, EMB + NF_EMB), lambda i: (i, 0)),
        out_shape=jax.ShapeDtypeStruct((B, EMB + NF_EMB), jnp.float32),
    )(emb_rows, nf_rows, lin_W, lin_b)


def kernel(node_feats, emb_weight, lin_W, lin_b, batch):
    emb_rows, nf_rows = _sc_gather2(emb_weight, node_feats, batch)
    return _tc_fuse(emb_rows, nf_rows, lin_W, lin_b.reshape(1, NF_EMB))


# speedup vs baseline: 3.7308x; 1.3337x over previous
import functools

import jax
import jax.numpy as jnp
from jax import lax
from jax.experimental import pallas as pl
from jax.experimental.pallas import tpu as pltpu
from jax.experimental.pallas import tpu_sc as plsc

N = 100000
D_FEAT = 128
EMB = 128
NF_EMB = 64
B = 16384

NUM_CORES = 2
NUM_SUBCORES = 16
NUM_WORKERS = NUM_CORES * NUM_SUBCORES
B_PER_W = B // NUM_WORKERS


def _sc_gather2(emb_weight, node_feats, batch):
    mesh = plsc.VectorSubcoreMesh(core_axis_name="c", subcore_axis_name="s")

    @functools.partial(
        pl.kernel,
        mesh=mesh,
        out_type=(
            jax.ShapeDtypeStruct((B, EMB), jnp.float32),
            jax.ShapeDtypeStruct((B, D_FEAT), jnp.float32),
        ),
        scratch_types=[
            pltpu.VMEM((B_PER_W,), jnp.int32),
            pltpu.VMEM((B_PER_W, EMB), jnp.float32),
            pltpu.SemaphoreType.DMA,
        ],
    )
    def k(emb_hbm, nf_hbm, idx_hbm, emb_out, nf_out, idx_v, rows_v, sem):
        wid = lax.axis_index("s") * NUM_CORES + lax.axis_index("c")
        base = wid * B_PER_W
        pltpu.sync_copy(idx_hbm.at[pl.ds(base, B_PER_W)], idx_v)
        pltpu.async_copy(emb_hbm.at[idx_v], rows_v, sem).wait()
        pltpu.sync_copy(rows_v, emb_out.at[pl.ds(base, B_PER_W)])
        pltpu.async_copy(nf_hbm.at[idx_v], rows_v, sem).wait()
        pltpu.sync_copy(rows_v, nf_out.at[pl.ds(base, B_PER_W)])

    return k(emb_weight, node_feats, batch)


def _tc_fuse_kernel(emb_ref, nf_ref, w_ref, b_ref, out_ref):
    zT = jax.lax.dot_general(
        w_ref[...], nf_ref[...],
        dimension_numbers=(((0,), (1,)), ((), ())),
        preferred_element_type=jnp.float32,
    )
    out_ref[:EMB, :] = emb_ref[...].T
    out_ref[EMB:, :] = zT + b_ref[...]


def _tc_fuse(emb_rows, nf_rows, lin_W, lin_b):
    bn = 2048
    grid = (B // bn,)
    return pl.pallas_call(
        _tc_fuse_kernel,
        grid=grid,
        in_specs=[
            pl.BlockSpec((bn, EMB), lambda i: (i, 0)),
            pl.BlockSpec((bn, D_FEAT), lambda i: (i, 0)),
            pl.BlockSpec((D_FEAT, NF_EMB), lambda i: (0, 0)),
            pl.BlockSpec((NF_EMB, 1), lambda i: (0, 0)),
        ],
        out_specs=pl.BlockSpec((EMB + NF_EMB, bn), lambda i: (0, i)),
        out_shape=jax.ShapeDtypeStruct((EMB + NF_EMB, B), jnp.float32),
    )(emb_rows, nf_rows, lin_W, lin_b)


def kernel(node_feats, emb_weight, lin_W, lin_b, batch):
    emb_rows, nf_rows = _sc_gather2(emb_weight, node_feats, batch)
    return _tc_fuse(emb_rows, nf_rows, lin_W, lin_b.reshape(NF_EMB, 1)).T


# speedup vs baseline: 3.8473x; 1.0312x over previous
import functools

import jax
import jax.numpy as jnp
from jax import lax
from jax.experimental import pallas as pl
from jax.experimental.pallas import tpu as pltpu
from jax.experimental.pallas import tpu_sc as plsc

N = 100000
D_FEAT = 128
EMB = 128
NF_EMB = 64
B = 16384

NUM_CORES = 2
NUM_SUBCORES = 16
NUM_WORKERS = NUM_CORES * NUM_SUBCORES
B_PER_W = B // NUM_WORKERS


CHUNK = B_PER_W // 2


def _sc_gather2(emb_weight, node_feats, batch):
    mesh = plsc.VectorSubcoreMesh(core_axis_name="c", subcore_axis_name="s")

    @functools.partial(
        pl.kernel,
        mesh=mesh,
        out_type=(
            jax.ShapeDtypeStruct((B, EMB), jnp.float32),
            jax.ShapeDtypeStruct((B, D_FEAT), jnp.float32),
        ),
        scratch_types=[
            pltpu.VMEM((B_PER_W,), jnp.int32),
            pltpu.VMEM((CHUNK, EMB), jnp.float32),
            pltpu.VMEM((CHUNK, EMB), jnp.float32),
            pltpu.SemaphoreType.DMA,
            pltpu.SemaphoreType.DMA,
            pltpu.SemaphoreType.DMA,
            pltpu.SemaphoreType.DMA,
        ],
    )
    def k(emb_hbm, nf_hbm, idx_hbm, emb_out, nf_out,
          idx_v, buf0, buf1, g0, g1, w0, w1):
        wid = lax.axis_index("s") * NUM_CORES + lax.axis_index("c")
        base = wid * B_PER_W
        pltpu.sync_copy(idx_hbm.at[pl.ds(base, B_PER_W)], idx_v)
        i0 = idx_v.at[pl.ds(0, CHUNK)]
        i1 = idx_v.at[pl.ds(CHUNK, CHUNK)]
        h0 = pltpu.async_copy(emb_hbm.at[i0], buf0, g0)
        h1 = pltpu.async_copy(nf_hbm.at[i0], buf1, g1)
        h0.wait()
        pltpu.async_copy(buf0, emb_out.at[pl.ds(base, CHUNK)], w0)
        h1.wait()
        pltpu.async_copy(buf1, nf_out.at[pl.ds(base, CHUNK)], w1)
        pltpu.make_async_copy(buf0, emb_out.at[pl.ds(base, CHUNK)], w0).wait()
        h2 = pltpu.async_copy(emb_hbm.at[i1], buf0, g0)
        pltpu.make_async_copy(buf1, nf_out.at[pl.ds(base, CHUNK)], w1).wait()
        h3 = pltpu.async_copy(nf_hbm.at[i1], buf1, g1)
        h2.wait()
        pltpu.async_copy(buf0, emb_out.at[pl.ds(base + CHUNK, CHUNK)], w0)
        h3.wait()
        pltpu.async_copy(buf1, nf_out.at[pl.ds(base + CHUNK, CHUNK)], w1)
        pltpu.make_async_copy(
            buf0, emb_out.at[pl.ds(base + CHUNK, CHUNK)], w0).wait()
        pltpu.make_async_copy(
            buf1, nf_out.at[pl.ds(base + CHUNK, CHUNK)], w1).wait()

    return k(emb_weight, node_feats, batch)


def _tc_fuse_kernel(emb_ref, nf_ref, w_ref, b_ref, out_ref):
    zT = jax.lax.dot_general(
        w_ref[...], nf_ref[...],
        dimension_numbers=(((0,), (1,)), ((), ())),
        preferred_element_type=jnp.float32,
    )
    out_ref[:EMB, :] = emb_ref[...].T
    out_ref[EMB:, :] = zT + b_ref[...]


def _tc_fuse(emb_rows, nf_rows, lin_W, lin_b):
    bn = 4096
    grid = (B // bn,)
    return pl.pallas_call(
        _tc_fuse_kernel,
        grid=grid,
        in_specs=[
            pl.BlockSpec((bn, EMB), lambda i: (i, 0)),
            pl.BlockSpec((bn, D_FEAT), lambda i: (i, 0)),
            pl.BlockSpec((D_FEAT, NF_EMB), lambda i: (0, 0)),
            pl.BlockSpec((NF_EMB, 1), lambda i: (0, 0)),
        ],
        out_specs=pl.BlockSpec((EMB + NF_EMB, bn), lambda i: (0, i)),
        out_shape=jax.ShapeDtypeStruct((EMB + NF_EMB, B), jnp.float32),
    )(emb_rows, nf_rows, lin_W, lin_b)


def kernel(node_feats, emb_weight, lin_W, lin_b, batch):
    emb_rows, nf_rows = _sc_gather2(emb_weight, node_feats, batch)
    return _tc_fuse(emb_rows, nf_rows, lin_W, lin_b.reshape(NF_EMB, 1)).T
